# Initial kernel scaffold; baseline (speedup 1.0000x reference)
import jax
import jax.numpy as jnp
from jax.experimental import pallas as pl


def kernel(x, edge_index, edge_weight, h0, W_z, b_z, W_r, b_r, W_h, b_h, Lz_W, Lz_b, Lr_W, Lr_b, Lh_W, Lh_b, W_out, b_out):
    raise NotImplementedError("write your pallas kernel here")


# speedup vs baseline: 24.4522x; 24.4522x over previous
import jax
import jax.numpy as jnp
from jax import lax
from jax.experimental import pallas as pl
from jax.experimental.pallas import tpu as pltpu
from jax.experimental.pallas import tpu_sc as plsc

NC, NS, L = 2, 16, 16
CH = 128


def _deg_kernel(col_p, ew_p, *, e_pad, n_pad):
    ept = e_pad // (NC * NS)
    stripe = n_pad // NS

    mesh = plsc.VectorSubcoreMesh(core_axis_name="c", subcore_axis_name="s",
                                  num_cores=NC, num_subcores=NS)

    def body(col_hbm, ew_hbm, degp_hbm, col_v, ew_v, deg_v, buf_v, deg_sh):
        cid = lax.axis_index("c")
        sid = lax.axis_index("s")
        tid = cid * NS + sid
        base = tid * ept
        pltpu.sync_copy(col_hbm.at[pl.ds(base, ept)], col_v)
        pltpu.sync_copy(ew_hbm.at[pl.ds(base, ept)], ew_v)
        zero = jnp.zeros((L,), jnp.float32)

        def z(i, c):
            deg_v[pl.ds(i * L, L)] = zero
            return c
        lax.fori_loop(0, n_pad // L, z, 0)

        def acc(i, c):
            c16 = col_v[pl.ds(i * L, L)]
            w16 = ew_v[pl.ds(i * L, L)]
            plsc.addupdate_scatter(deg_v, [c16], w16)
            return c
        lax.fori_loop(0, ept // L, acc, 0)

        pltpu.sync_copy(deg_v, deg_sh.at[sid])
        plsc.subcore_barrier()
        for r in range(NS):
            pltpu.sync_copy(deg_sh.at[r, pl.ds(sid * stripe, stripe)],
                            buf_v.at[r])
        def red(i, c):
            s = buf_v[0, pl.ds(i * L, L)]
            for r in range(1, NS):
                s = s + buf_v[r, pl.ds(i * L, L)]
            deg_v[pl.ds(i * L, L)] = s
            return c
        lax.fori_loop(0, stripe // L, red, 0)
        pltpu.sync_copy(deg_v.at[pl.ds(0, stripe)],
                        degp_hbm.at[cid, pl.ds(sid * stripe, stripe)])

    k = pl.kernel(
        body,
        out_type=jax.ShapeDtypeStruct((NC, n_pad), jnp.float32),
        mesh=mesh,
        compiler_params=pltpu.CompilerParams(needs_layout_passes=False),
        scratch_types=[
            pltpu.VMEM((ept,), jnp.int32),
            pltpu.VMEM((ept,), jnp.float32),
            pltpu.VMEM((n_pad,), jnp.float32),
            pltpu.VMEM((NS, stripe), jnp.float32),
            pltpu.VMEM_SHARED((NS, n_pad), jnp.float32),
        ],
    )
    return k(col_p, ew_p)


def _dis_kernel(degp2d):
    half = degp2d.shape[0] // 2

    def body(d_ref, o_ref):
        deg = d_ref[0:half, :] + d_ref[half:2 * half, :] + 2.0
        o_ref[...] = lax.rsqrt(deg)

    return pl.pallas_call(
        body,
        out_shape=jax.ShapeDtypeStruct((half, 128), jnp.float32),
    )(degp2d)


def _spmm_kernel(xf, row_p, col3d, ew_p, dis_p, *, b, n, f, e_pad, n_pad):
    ept = e_pad // NS
    nchunks = ept // CH
    nb = b // NC
    srows = n_pad // NS
    frows = 128
    assert srows % frows == 0

    mesh = plsc.VectorSubcoreMesh(core_axis_name="c", subcore_axis_name="s",
                                  num_cores=NC, num_subcores=NS)

    zrows = 16

    def body(x_hbm, row_hbm, col_hbm, ew_hbm, dis_hbm, acc_hbm,
             rowbuf, colbuf, vbuf, dis_v, gidx_v, gbuf, zbuf, acc_sh):
        cid = lax.axis_index("c")
        sid = lax.axis_index("s")
        base = sid * ept
        pltpu.sync_copy(dis_hbm, dis_v)

        zero = jnp.zeros((L,), jnp.float32)
        def zz(i, c):
            for g in range(f // L):
                zbuf[i, pl.ds(g * L, L)] = zero
            return c
        lax.fori_loop(0, zrows, zz, 0)

        for bb in range(nb):
            bat = cid * nb + bb
            boff = bat * n
            poff = bat * n_pad
            for k in range(srows // zrows):
                pltpu.sync_copy(
                    zbuf, acc_sh.at[pl.ds(sid * srows + k * zrows, zrows)])
            plsc.subcore_barrier()

            def chunk(c, carry):
                eoff = base + c * CH
                pltpu.sync_copy(row_hbm.at[pl.ds(eoff, CH)], rowbuf)
                pltpu.sync_copy(col_hbm.at[sid].at[pl.ds(c, 1)], colbuf)
                pltpu.sync_copy(ew_hbm.at[pl.ds(eoff, CH)], vbuf)
                for g in range(CH // L):
                    r16 = rowbuf[pl.ds(g * L, L)]
                    d16 = plsc.load_gather(dis_v, [r16])
                    vbuf[pl.ds(g * L, L)] = vbuf[pl.ds(g * L, L)] * d16
                    gidx_v[pl.ds(g * L, L)] = r16 + boff
                pltpu.sync_copy(x_hbm.at[gidx_v], gbuf)

                def scale(e, cc):
                    wb = plsc.load_gather(
                        vbuf, [jnp.zeros((L,), jnp.int32) + e])
                    for g in range(f // L):
                        gbuf[e, pl.ds(g * L, L)] = (
                            gbuf[e, pl.ds(g * L, L)] * wb)
                    return cc
                lax.fori_loop(0, CH, scale, 0)
                pltpu.sync_copy(gbuf, acc_sh.at[colbuf.at[0]], add=True)
                return carry
            lax.fori_loop(0, nchunks, chunk, 0)
            plsc.subcore_barrier()

            for k in range(srows // frows):
                off = sid * srows + k * frows
                pltpu.sync_copy(acc_sh.at[pl.ds(off, frows)],
                                acc_hbm.at[pl.ds(poff + off, frows)])

    k = pl.kernel(
        body,
        out_type=jax.ShapeDtypeStruct((b * n_pad, f), jnp.float32),
        mesh=mesh,
        compiler_params=pltpu.CompilerParams(needs_layout_passes=False),
        scratch_types=[
            pltpu.VMEM((CH,), jnp.int32),
            pltpu.VMEM((1, CH), jnp.int32),
            pltpu.VMEM((CH,), jnp.float32),
            pltpu.VMEM((n_pad,), jnp.float32),
            pltpu.VMEM((CH,), jnp.int32),
            pltpu.VMEM((CH, f), jnp.float32),
            pltpu.VMEM((zrows, f), jnp.float32),
            pltpu.VMEM_SHARED((n_pad, f), jnp.float32),
        ],
    )
    return k(xf, row_p, col3d, ew_p, dis_p)


def _dense_kernel(disb, xf, accf, h0f,
                  W_z, b_z, W_r, b_r, W_h, b_h,
                  Lz_W, Lz_b, Lr_W, Lr_b, Lh_W, Lh_b,
                  W_out, b_out, *, blk):
    rows, f = xf.shape
    dh = h0f.shape[1]
    dd = W_out.shape[1]
    grid = rows // blk

    def body(dis_r, x_r, acc_r, h0_r,
             Wz_r, bz_r, Wr_r, br_r, Wh_r, bh_r,
             Lz_r, Lzb_r, Lr_r, Lrb_r, Lh_r, Lhb_r,
             Wo_r, bo_r, H_r, y_r):
        dis = dis_r[...]
        x = x_r[...]
        acc = acc_r[...]
        h0 = h0_r[...]
        agg = dis * acc + (2.0 * dis * dis) * x
        dot = lambda a, w: jnp.dot(a, w, preferred_element_type=jnp.float32)
        Cz = dot(agg, Wz_r[...]) + bz_r[...]
        Cr = dot(agg, Wr_r[...]) + br_r[...]
        Ch = dot(agg, Wh_r[...]) + bh_r[...]
        Z = jax.nn.sigmoid(dot(Cz, Lz_r[0:dh, :]) + dot(h0, Lz_r[dh:2 * dh, :])
                           + Lzb_r[...])
        R = jax.nn.sigmoid(dot(Cr, Lr_r[0:dh, :]) + dot(h0, Lr_r[dh:2 * dh, :])
                           + Lrb_r[...])
        Ht = jnp.tanh(dot(Ch, Lh_r[0:dh, :]) + dot(h0 * R, Lh_r[dh:2 * dh, :])
                      + Lhb_r[...])
        H = Z * h0 + (1.0 - Z) * Ht
        H_r[...] = H
        y_r[...] = dot(jnp.maximum(H, 0.0), Wo_r[...]) + bo_r[...]

    full = lambda shape: pl.BlockSpec(shape, lambda i: (0,) * len(shape))
    out = pl.pallas_call(
        body,
        grid=(grid,),
        in_specs=[
            pl.BlockSpec((blk, 1), lambda i: (i, 0)),
            pl.BlockSpec((blk, f), lambda i: (i, 0)),
            pl.BlockSpec((blk, f), lambda i: (i, 0)),
            pl.BlockSpec((blk, dh), lambda i: (i, 0)),
            full((f, dh)), full((1, dh)),
            full((f, dh)), full((1, dh)),
            full((f, dh)), full((1, dh)),
            full((2 * dh, dh)), full((1, dh)),
            full((2 * dh, dh)), full((1, dh)),
            full((2 * dh, dh)), full((1, dh)),
            full((dh, dd)), full((1, dd)),
        ],
        out_specs=[
            pl.BlockSpec((blk, dh), lambda i: (i, 0)),
            pl.BlockSpec((blk, dd), lambda i: (i, 0)),
        ],
        out_shape=[
            jax.ShapeDtypeStruct((rows, dh), jnp.float32),
            jax.ShapeDtypeStruct((rows, dd), jnp.float32),
        ],
    )(disb, xf, accf, h0f,
      W_z, b_z.reshape(1, dh), W_r, b_r.reshape(1, dh), W_h, b_h.reshape(1, dh),
      Lz_W, Lz_b.reshape(1, dh), Lr_W, Lr_b.reshape(1, dh),
      Lh_W, Lh_b.reshape(1, dh),
      W_out, b_out.reshape(1, dd))
    return out


def kernel(x, edge_index, edge_weight, h0,
           W_z, b_z, W_r, b_r, W_h, b_h,
           Lz_W, Lz_b, Lr_W, Lr_b, Lh_W, Lh_b,
           W_out, b_out):
    b, n, f = x.shape
    dh = h0.shape[2]
    e = edge_weight.shape[0]

    e_pad = ((e + NC * NS * CH - 1) // (NC * NS * CH)) * (NC * NS * CH)
    n_pad = ((n + NS * CH - 1) // (NS * CH)) * (NS * CH)
    pad = e_pad - e
    row_p = jnp.concatenate([edge_index[0], jnp.zeros((pad,), jnp.int32)])
    col_p = jnp.concatenate([edge_index[1], jnp.zeros((pad,), jnp.int32)])
    ew_p = jnp.concatenate([edge_weight, jnp.zeros((pad,), jnp.float32)])
    col3d = col_p.reshape(NS, e_pad // NS // CH, CH)

    degp = _deg_kernel(col_p, ew_p, e_pad=e_pad, n_pad=n_pad)
    dis2d = _dis_kernel(degp.reshape(2 * n_pad // 128, 128))
    dis_p = dis2d.reshape(n_pad)

    xf = x.reshape(b * n, f)
    accp = _spmm_kernel(xf, row_p, col3d, ew_p, dis_p,
                        b=b, n=n, f=f, e_pad=e_pad, n_pad=n_pad)
    accf = accp.reshape(b, n_pad, f)[:, :n, :].reshape(b * n, f)

    dis_n = dis_p[:n]
    disb = jnp.broadcast_to(dis_n[None, :], (b, n)).reshape(b * n, 1)
    h0f = h0.reshape(b * n, dh)
    H, y = _dense_kernel(disb, xf, accf, h0f,
                         W_z, b_z, W_r, b_r, W_h, b_h,
                         Lz_W, Lz_b, Lr_W, Lr_b, Lh_W, Lh_b,
                         W_out, b_out, blk=2000)
    return (H.reshape(b, n, dh), y.reshape(b, n, W_out.shape[1]))


# speedup vs baseline: 39.1732x; 1.6020x over previous
import jax
import jax.numpy as jnp
from jax import lax
from jax.experimental import pallas as pl
from jax.experimental.pallas import tpu as pltpu
from jax.experimental.pallas import tpu_sc as plsc

NC, NS, L = 2, 16, 16
CH = 128


def _deg_kernel(col_p, ew_p, *, e_pad, n_pad):
    ept = e_pad // (NC * NS)
    stripe = n_pad // NS

    mesh = plsc.VectorSubcoreMesh(core_axis_name="c", subcore_axis_name="s",
                                  num_cores=NC, num_subcores=NS)

    def body(col_hbm, ew_hbm, degp_hbm, col_v, ew_v, deg_v, buf_v, deg_sh):
        cid = lax.axis_index("c")
        sid = lax.axis_index("s")
        tid = cid * NS + sid
        base = tid * ept
        pltpu.sync_copy(col_hbm.at[pl.ds(base, ept)], col_v)
        pltpu.sync_copy(ew_hbm.at[pl.ds(base, ept)], ew_v)
        zero = jnp.zeros((L,), jnp.float32)

        def z(i, c):
            deg_v[pl.ds(i * L, L)] = zero
            return c
        lax.fori_loop(0, n_pad // L, z, 0)

        def acc(i, c):
            c16 = col_v[pl.ds(i * L, L)]
            w16 = ew_v[pl.ds(i * L, L)]
            plsc.addupdate_scatter(deg_v, [c16], w16)
            return c
        lax.fori_loop(0, ept // L, acc, 0)

        pltpu.sync_copy(deg_v, deg_sh.at[sid])
        plsc.subcore_barrier()
        for r in range(NS):
            pltpu.sync_copy(deg_sh.at[r, pl.ds(sid * stripe, stripe)],
                            buf_v.at[r])
        def red(i, c):
            s = buf_v[0, pl.ds(i * L, L)]
            for r in range(1, NS):
                s = s + buf_v[r, pl.ds(i * L, L)]
            deg_v[pl.ds(i * L, L)] = s
            return c
        lax.fori_loop(0, stripe // L, red, 0)
        pltpu.sync_copy(deg_v.at[pl.ds(0, stripe)],
                        degp_hbm.at[cid, pl.ds(sid * stripe, stripe)])

    k = pl.kernel(
        body,
        out_type=jax.ShapeDtypeStruct((NC, n_pad), jnp.float32),
        mesh=mesh,
        compiler_params=pltpu.CompilerParams(needs_layout_passes=False),
        scratch_types=[
            pltpu.VMEM((ept,), jnp.int32),
            pltpu.VMEM((ept,), jnp.float32),
            pltpu.VMEM((n_pad,), jnp.float32),
            pltpu.VMEM((NS, stripe), jnp.float32),
            pltpu.VMEM_SHARED((NS, n_pad), jnp.float32),
        ],
    )
    return k(col_p, ew_p)


def _dis_kernel(degp2d):
    half = degp2d.shape[0] // 2

    def body(d_ref, o_ref):
        deg = d_ref[0:half, :] + d_ref[half:2 * half, :] + 2.0
        o_ref[...] = lax.rsqrt(deg)

    return pl.pallas_call(
        body,
        out_shape=jax.ShapeDtypeStruct((half, 128), jnp.float32),
    )(degp2d)


def _spmm_kernel(xf, row_p, col3d, ew_p, dis_p, *, b, n, f, e_pad, n_pad):
    ept = e_pad // NS
    nchunks = ept // CH
    nb = b // NC
    srows = n_pad // NS
    frows = 128
    assert srows % frows == 0

    mesh = plsc.VectorSubcoreMesh(core_axis_name="c", subcore_axis_name="s",
                                  num_cores=NC, num_subcores=NS)

    G = f // L
    assert nchunks % 4 == 0
    quads = nchunks // 4

    def body(x_hbm, row_hbm, col_hbm, ew_hbm, dis_hbm, acc_hbm,
             rows4, cols4, vs4, gidx2, gbuf2, dis_v, acc_sh,
             es0, es1, es2, es3, gs0, gs1, ss0, ss1):
        esems = [es0, es1, es2, es3]
        gsems = [gs0, gs1]
        ssems = [ss0, ss1]
        cid = lax.axis_index("c")
        sid = lax.axis_index("s")
        base = sid * ept
        pltpu.sync_copy(dis_hbm, dis_v)
        zero = jnp.zeros((L,), jnp.float32)

        def edge_copies(cc, s4):
            eoff = base + cc * CH
            return [
                pltpu.make_async_copy(row_hbm.at[pl.ds(eoff, CH)],
                                      rows4.at[s4], esems[s4]),
                pltpu.make_async_copy(col_hbm.at[sid].at[pl.ds(cc, 1)],
                                      cols4.at[pl.ds(s4, 1)], esems[s4]),
                pltpu.make_async_copy(ew_hbm.at[pl.ds(eoff, CH)],
                                      vs4.at[s4], esems[s4]),
            ]

        def issue_edges(cc, s4):
            for c_ in edge_copies(cc, s4):
                c_.start()

        def wait_edges(cc, s4):
            for c_ in edge_copies(cc, s4):
                c_.wait()

        def prep(cc, s2, s4, boff):
            for g in range(G):
                r16 = rows4[s4, pl.ds(g * L, L)]
                d16 = plsc.load_gather(dis_v, [r16])
                vs4[s4, pl.ds(g * L, L)] = vs4[s4, pl.ds(g * L, L)] * d16
                gidx2[s2, pl.ds(g * L, L)] = r16 + boff

        def gather_copy(s2):
            return pltpu.make_async_copy(x_hbm.at[gidx2.at[s2]],
                                         gbuf2.at[s2], gsems[s2])

        def scatter_copy(s2, s4):
            return pltpu.make_async_copy(gbuf2.at[s2],
                                         acc_sh.at[cols4.at[s4]], ssems[s2])

        def scale(s2, s4):
            def sc(e, carry):
                wb = plsc.load_gather(vs4.at[s4],
                                      [jnp.zeros((L,), jnp.int32) + e])
                for g in range(G):
                    gbuf2[s2, e, pl.ds(g * L, L)] = (
                        gbuf2[s2, e, pl.ds(g * L, L)] * wb)
                return carry
            lax.fori_loop(0, CH, sc, 0)

        for bb in range(nb):
            bat = cid * nb + bb
            boff = bat * n
            poff = bat * n_pad
            def zz(i, c):
                for g in range(G):
                    gbuf2[0, i, pl.ds(g * L, L)] = zero
                return c
            lax.fori_loop(0, CH, zz, 0)
            nzc = srows // CH
            for k in range(nzc):
                pltpu.async_copy(gbuf2.at[0],
                                 acc_sh.at[pl.ds(sid * srows + k * CH, CH)],
                                 ss0)
            for k in range(nzc):
                pltpu.make_async_copy(
                    gbuf2.at[0],
                    acc_sh.at[pl.ds(sid * srows + k * CH, CH)], ss0).wait()
            plsc.subcore_barrier()

            issue_edges(0, 0)
            issue_edges(1, 1)
            issue_edges(2, 2)
            wait_edges(0, 0)
            prep(0, 0, 0, boff)
            gather_copy(0).start()

            def quad(q, carry):
                for u in range(4):
                    cc = q * 4 + u
                    s2, s4 = u % 2, u % 4
                    n2, n4 = (u + 1) % 2, (u + 1) % 4
                    p4 = (u + 3) % 4

                    def prep_next():
                        wait_edges(cc + 1, n4)
                        prep(cc + 1, n2, n4, boff)

                    def free_prev():
                        scatter_copy(n2, p4).wait()

                    def start_next():
                        gather_copy(n2).start()

                    def fetch_ahead():
                        issue_edges(cc + 3, p4)

                    if u == 3:
                        @pl.when(q < quads - 1)
                        def _():
                            prep_next()
                            free_prev()
                            start_next()
                            fetch_ahead()
                    elif u == 0:
                        prep_next()
                        @pl.when(q > 0)
                        def _():
                            free_prev()
                        start_next()
                        fetch_ahead()
                    else:
                        prep_next()
                        free_prev()
                        start_next()
                        @pl.when(q < quads - 1)
                        def _():
                            fetch_ahead()

                    gather_copy(s2).wait()
                    scale(s2, s4)
                    pltpu.async_copy(gbuf2.at[s2], acc_sh.at[cols4.at[s4]],
                                     ssems[s2], add=True)
                return carry
            lax.fori_loop(0, quads, quad, 0)
            scatter_copy(0, 2).wait()
            scatter_copy(1, 3).wait()
            plsc.subcore_barrier()

            for k in range(srows // frows):
                off = sid * srows + k * frows
                pltpu.async_copy(acc_sh.at[pl.ds(off, frows)],
                                 acc_hbm.at[pl.ds(poff + off, frows)], es0)
            for k in range(srows // frows):
                pltpu.make_async_copy(
                    acc_sh.at[pl.ds(sid * srows + k * frows, frows)],
                    acc_hbm.at[pl.ds(poff + sid * srows + k * frows, frows)],
                    es0).wait()

    k = pl.kernel(
        body,
        out_type=jax.ShapeDtypeStruct((b * n_pad, f), jnp.float32),
        mesh=mesh,
        compiler_params=pltpu.CompilerParams(needs_layout_passes=False),
        scratch_types=[
            pltpu.VMEM((4, CH), jnp.int32),
            pltpu.VMEM((4, CH), jnp.int32),
            pltpu.VMEM((4, CH), jnp.float32),
            pltpu.VMEM((2, CH), jnp.int32),
            pltpu.VMEM((2, CH, f), jnp.float32),
            pltpu.VMEM((n_pad,), jnp.float32),
            pltpu.VMEM_SHARED((n_pad, f), jnp.float32),
            pltpu.SemaphoreType.DMA,
            pltpu.SemaphoreType.DMA,
            pltpu.SemaphoreType.DMA,
            pltpu.SemaphoreType.DMA,
            pltpu.SemaphoreType.DMA,
            pltpu.SemaphoreType.DMA,
            pltpu.SemaphoreType.DMA,
            pltpu.SemaphoreType.DMA,
        ],
    )
    return k(xf, row_p, col3d, ew_p, dis_p)


def _dense_kernel(disb, xf, accf, h0f,
                  W_z, b_z, W_r, b_r, W_h, b_h,
                  Lz_W, Lz_b, Lr_W, Lr_b, Lh_W, Lh_b,
                  W_out, b_out, *, blk):
    rows, f = xf.shape
    dh = h0f.shape[1]
    dd = W_out.shape[1]
    grid = rows // blk

    def body(dis_r, x_r, acc_r, h0_r,
             Wz_r, bz_r, Wr_r, br_r, Wh_r, bh_r,
             Lz_r, Lzb_r, Lr_r, Lrb_r, Lh_r, Lhb_r,
             Wo_r, bo_r, H_r, y_r):
        dis = dis_r[...]
        x = x_r[...]
        acc = acc_r[...]
        h0 = h0_r[...]
        agg = dis * acc + (2.0 * dis * dis) * x
        dot = lambda a, w: jnp.dot(a, w, preferred_element_type=jnp.float32)
        Cz = dot(agg, Wz_r[...]) + bz_r[...]
        Cr = dot(agg, Wr_r[...]) + br_r[...]
        Ch = dot(agg, Wh_r[...]) + bh_r[...]
        Z = jax.nn.sigmoid(dot(Cz, Lz_r[0:dh, :]) + dot(h0, Lz_r[dh:2 * dh, :])
                           + Lzb_r[...])
        R = jax.nn.sigmoid(dot(Cr, Lr_r[0:dh, :]) + dot(h0, Lr_r[dh:2 * dh, :])
                           + Lrb_r[...])
        Ht = jnp.tanh(dot(Ch, Lh_r[0:dh, :]) + dot(h0 * R, Lh_r[dh:2 * dh, :])
                      + Lhb_r[...])
        H = Z * h0 + (1.0 - Z) * Ht
        H_r[...] = H
        y_r[...] = dot(jnp.maximum(H, 0.0), Wo_r[...]) + bo_r[...]

    full = lambda shape: pl.BlockSpec(shape, lambda i: (0,) * len(shape))
    out = pl.pallas_call(
        body,
        grid=(grid,),
        in_specs=[
            pl.BlockSpec((blk, 1), lambda i: (i, 0)),
            pl.BlockSpec((blk, f), lambda i: (i, 0)),
            pl.BlockSpec((blk, f), lambda i: (i, 0)),
            pl.BlockSpec((blk, dh), lambda i: (i, 0)),
            full((f, dh)), full((1, dh)),
            full((f, dh)), full((1, dh)),
            full((f, dh)), full((1, dh)),
            full((2 * dh, dh)), full((1, dh)),
            full((2 * dh, dh)), full((1, dh)),
            full((2 * dh, dh)), full((1, dh)),
            full((dh, dd)), full((1, dd)),
        ],
        out_specs=[
            pl.BlockSpec((blk, dh), lambda i: (i, 0)),
            pl.BlockSpec((blk, dd), lambda i: (i, 0)),
        ],
        out_shape=[
            jax.ShapeDtypeStruct((rows, dh), jnp.float32),
            jax.ShapeDtypeStruct((rows, dd), jnp.float32),
        ],
    )(disb, xf, accf, h0f,
      W_z, b_z.reshape(1, dh), W_r, b_r.reshape(1, dh), W_h, b_h.reshape(1, dh),
      Lz_W, Lz_b.reshape(1, dh), Lr_W, Lr_b.reshape(1, dh),
      Lh_W, Lh_b.reshape(1, dh),
      W_out, b_out.reshape(1, dd))
    return out


def kernel(x, edge_index, edge_weight, h0,
           W_z, b_z, W_r, b_r, W_h, b_h,
           Lz_W, Lz_b, Lr_W, Lr_b, Lh_W, Lh_b,
           W_out, b_out):
    b, n, f = x.shape
    dh = h0.shape[2]
    e = edge_weight.shape[0]

    e_pad = ((e + NC * NS * CH - 1) // (NC * NS * CH)) * (NC * NS * CH)
    n_pad = ((n + NS * CH - 1) // (NS * CH)) * (NS * CH)
    pad = e_pad - e
    row_p = jnp.concatenate([edge_index[0], jnp.zeros((pad,), jnp.int32)])
    col_p = jnp.concatenate([edge_index[1], jnp.zeros((pad,), jnp.int32)])
    ew_p = jnp.concatenate([edge_weight, jnp.zeros((pad,), jnp.float32)])
    col3d = col_p.reshape(NS, e_pad // NS // CH, CH)

    degp = _deg_kernel(col_p, ew_p, e_pad=e_pad, n_pad=n_pad)
    dis2d = _dis_kernel(degp.reshape(2 * n_pad // 128, 128))
    dis_p = dis2d.reshape(n_pad)

    xf = x.reshape(b * n, f)
    accp = _spmm_kernel(xf, row_p, col3d, ew_p, dis_p,
                        b=b, n=n, f=f, e_pad=e_pad, n_pad=n_pad)
    accf = accp.reshape(b, n_pad, f)[:, :n, :].reshape(b * n, f)

    dis_n = dis_p[:n]
    disb = jnp.broadcast_to(dis_n[None, :], (b, n)).reshape(b * n, 1)
    h0f = h0.reshape(b * n, dh)
    H, y = _dense_kernel(disb, xf, accf, h0f,
                         W_z, b_z, W_r, b_r, W_h, b_h,
                         Lz_W, Lz_b, Lr_W, Lr_b, Lh_W, Lh_b,
                         W_out, b_out, blk=2000)
    return (H.reshape(b, n, dh), y.reshape(b, n, W_out.shape[1]))
